# baseline (speedup 1.0000x reference)
import jax
import jax.numpy as jnp
from jax.experimental import pallas as pl
from jax.experimental.pallas import tpu as pltpu

N = 2048
D = 1024
F = 2048
E = 8
BT = 256
G = (N * 2) // BT + E - 1
P = G * BT
FH = F // 2
NEG = -1e30


def _meta_body(x_ref, rw_ref, rb_ref,
               pos0_ref, pos1_ref, w0_ref, w1_ref, be_ref, xb_ref):
    x = x_ref[...]
    xb_ref[...] = x.astype(jnp.bfloat16)
    logits = jax.lax.dot_general(
        x, rw_ref[...], (((1,), (0,)), ((), ())),
        preferred_element_type=jnp.float32) + rb_ref[...]
    eio = jax.lax.broadcasted_iota(jnp.int32, (N, E), 1)
    m0 = jnp.max(logits, axis=1, keepdims=True)
    e0 = jnp.min(jnp.where(logits == m0, eio, E), axis=1, keepdims=True)
    l2 = jnp.where(eio == e0, NEG, logits)
    m1 = jnp.max(l2, axis=1, keepdims=True)
    e1 = jnp.min(jnp.where(l2 == m1, eio, E), axis=1, keepdims=True)
    w0 = 1.0 / (1.0 + jnp.exp(m1 - m0))
    w1 = 1.0 - w0

    oh0 = (eio == e0).astype(jnp.float32)
    oh1 = (eio == e1).astype(jnp.float32)
    s = oh0 + oh1

    rio = jax.lax.broadcasted_iota(jnp.int32, (N, N), 0)
    cio = jax.lax.broadcasted_iota(jnp.int32, (N, N), 1)
    tri = (rio > cio).astype(jnp.bfloat16)
    cum = jax.lax.dot_general(
        tri, s.astype(jnp.bfloat16), (((1,), (0,)), ((), ())),
        preferred_element_type=jnp.float32)

    counts = cum[N - 1:N, :] + s[N - 1:N, :]
    counts_i = counts.astype(jnp.int32)
    pc = (((counts_i + BT - 1) // BT) * BT).astype(jnp.float32)
    er = jax.lax.broadcasted_iota(jnp.int32, (E, E), 0)
    ec = jax.lax.broadcasted_iota(jnp.int32, (E, E), 1)
    mlt = (er < ec).astype(jnp.float32)
    po = jax.lax.dot_general(
        pc, mlt, (((1,), (0,)), ((), ())),
        preferred_element_type=jnp.float32)

    rank0 = jnp.sum(oh0 * cum, axis=1, keepdims=True)
    rank1 = jnp.sum(oh1 * cum, axis=1, keepdims=True)
    off0 = jnp.sum(oh0 * po, axis=1, keepdims=True)
    off1 = jnp.sum(oh1 * po, axis=1, keepdims=True)
    pos0_ref[...] = (off0 + rank0).astype(jnp.int32)
    pos1_ref[...] = (off1 + rank1).astype(jnp.int32)
    w0_ref[...] = w0
    w1_ref[...] = w1

    end = po + pc
    gio = jax.lax.broadcasted_iota(jnp.int32, (32, 1), 0)
    owned = (gio.astype(jnp.float32) * BT >= end)
    be = jnp.sum(owned.astype(jnp.int32), axis=1, keepdims=True)
    be_ref[...] = jnp.minimum(be, E - 1)


def _moe_body(be_ref, xb_ref, p0r_ref, p1r_ref,
              wg_ref, wu_ref, wd_ref, ys_ref):
    g = pl.program_id(0)
    base = g * BT

    pio_c = jax.lax.broadcasted_iota(jnp.int32, (BT, 1), 0) + base
    a0 = (p0r_ref[...] == pio_c)
    a1 = (p1r_ref[...] == pio_c)
    gath = a0.astype(jnp.bfloat16) + a1.astype(jnp.bfloat16)
    rows = jax.lax.dot_general(
        gath, xb_ref[...], (((1,), (0,)), ((), ())),
        preferred_element_type=jnp.float32).astype(jnp.bfloat16)

    y = jnp.zeros((BT, D), dtype=jnp.float32)
    for f in range(2):
        wg_h = wg_ref[0][:, f * FH:(f + 1) * FH].astype(jnp.bfloat16)
        wu_h = wu_ref[0][:, f * FH:(f + 1) * FH].astype(jnp.bfloat16)
        wd_h = wd_ref[0][f * FH:(f + 1) * FH, :].astype(jnp.bfloat16)
        gate = jax.lax.dot_general(
            rows, wg_h, (((1,), (0,)), ((), ())),
            preferred_element_type=jnp.float32)
        up = jax.lax.dot_general(
            rows, wu_h, (((1,), (0,)), ((), ())),
            preferred_element_type=jnp.float32)
        h = (gate * (1.0 / (1.0 + jnp.exp(-gate))) * up).astype(jnp.bfloat16)
        y = y + jax.lax.dot_general(
            h, wd_h, (((1,), (0,)), ((), ())),
            preferred_element_type=jnp.float32)

    ys_ref[...] = y.astype(jnp.bfloat16)


def _comb_body(ys_ref, p0c_ref, p1c_ref, w0c_ref, w1c_ref, out_ref):
    pio_r = jax.lax.broadcasted_iota(jnp.int32, (1, P), 1)
    a0 = (p0c_ref[...] == pio_r)
    a1 = (p1c_ref[...] == pio_r)
    w = (jnp.where(a0, w0c_ref[...], 0.0)
         + jnp.where(a1, w1c_ref[...], 0.0)).astype(jnp.bfloat16)
    out_ref[...] = jax.lax.dot_general(
        w, ys_ref[...], (((1,), (0,)), ((), ())),
        preferred_element_type=jnp.float32)


def kernel(x, router_w, router_b, w_gate, w_up, w_down):
    pos0, pos1, w0, w1, be, x_bf = pl.pallas_call(
        _meta_body,
        out_shape=[
            jax.ShapeDtypeStruct((N, 1), jnp.int32),
            jax.ShapeDtypeStruct((N, 1), jnp.int32),
            jax.ShapeDtypeStruct((N, 1), jnp.float32),
            jax.ShapeDtypeStruct((N, 1), jnp.float32),
            jax.ShapeDtypeStruct((32, 1), jnp.int32),
            jax.ShapeDtypeStruct((N, D), jnp.bfloat16),
        ],
        compiler_params=pltpu.CompilerParams(
            vmem_limit_bytes=128 * 1024 * 1024),
    )(x, router_w, router_b.reshape(1, E))

    be_flat = be.reshape(-1)[:G]
    pos0_r = pos0.reshape(1, N)
    pos1_r = pos1.reshape(1, N)

    moe_spec = pltpu.PrefetchScalarGridSpec(
        num_scalar_prefetch=1,
        grid=(G,),
        in_specs=[
            pl.BlockSpec((N, D), lambda g, be: (0, 0)),
            pl.BlockSpec((1, N), lambda g, be: (0, 0)),
            pl.BlockSpec((1, N), lambda g, be: (0, 0)),
            pl.BlockSpec((1, D, F), lambda g, be: (be[g], 0, 0)),
            pl.BlockSpec((1, D, F), lambda g, be: (be[g], 0, 0)),
            pl.BlockSpec((1, F, D), lambda g, be: (be[g], 0, 0)),
        ],
        out_specs=pl.BlockSpec((BT, D), lambda g, be: (g, 0)),
    )
    y_sorted = pl.pallas_call(
        _moe_body,
        grid_spec=moe_spec,
        out_shape=jax.ShapeDtypeStruct((P, D), jnp.bfloat16),
        compiler_params=pltpu.CompilerParams(
            dimension_semantics=("arbitrary",),
            vmem_limit_bytes=128 * 1024 * 1024),
    )(be_flat, x_bf, pos0_r, pos1_r, w_gate, w_up, w_down)

    out = pl.pallas_call(
        _comb_body,
        grid=(N // BT,),
        in_specs=[
            pl.BlockSpec((P, D), lambda t: (0, 0)),
            pl.BlockSpec((BT, 1), lambda t: (t, 0)),
            pl.BlockSpec((BT, 1), lambda t: (t, 0)),
            pl.BlockSpec((BT, 1), lambda t: (t, 0)),
            pl.BlockSpec((BT, 1), lambda t: (t, 0)),
        ],
        out_specs=pl.BlockSpec((BT, D), lambda t: (t, 0)),
        out_shape=jax.ShapeDtypeStruct((N, D), jnp.float32),
        compiler_params=pltpu.CompilerParams(
            dimension_semantics=("arbitrary",),
            vmem_limit_bytes=128 * 1024 * 1024),
    )(y_sorted, pos0, pos1, w0, w1)
    return out


# speedup vs baseline: 1.0016x; 1.0016x over previous
import jax
import jax.numpy as jnp
from jax.experimental import pallas as pl
from jax.experimental.pallas import tpu as pltpu

N = 2048
D = 1024
F = 2048
E = 8
BT = 256
G = (N * 2) // BT + E - 1
P = G * BT
FH = F // 2
NEG = -1e30


def _meta_body(x_ref, rw_ref, rb_ref,
               pos0_ref, pos1_ref, w0_ref, w1_ref, be_ref, xb_ref):
    x = x_ref[...]
    xb_ref[...] = x.astype(jnp.bfloat16)
    logits = jax.lax.dot_general(
        x, rw_ref[...], (((1,), (0,)), ((), ())),
        preferred_element_type=jnp.float32) + rb_ref[...]
    eio = jax.lax.broadcasted_iota(jnp.int32, (N, E), 1)
    m0 = jnp.max(logits, axis=1, keepdims=True)
    e0 = jnp.min(jnp.where(logits == m0, eio, E), axis=1, keepdims=True)
    l2 = jnp.where(eio == e0, NEG, logits)
    m1 = jnp.max(l2, axis=1, keepdims=True)
    e1 = jnp.min(jnp.where(l2 == m1, eio, E), axis=1, keepdims=True)
    w0 = 1.0 / (1.0 + jnp.exp(m1 - m0))
    w1 = 1.0 - w0

    oh0 = (eio == e0).astype(jnp.float32)
    oh1 = (eio == e1).astype(jnp.float32)
    s = oh0 + oh1

    rio = jax.lax.broadcasted_iota(jnp.int32, (N, N), 0)
    cio = jax.lax.broadcasted_iota(jnp.int32, (N, N), 1)
    tri = (rio > cio).astype(jnp.bfloat16)
    cum = jax.lax.dot_general(
        tri, s.astype(jnp.bfloat16), (((1,), (0,)), ((), ())),
        preferred_element_type=jnp.float32)

    counts = cum[N - 1:N, :] + s[N - 1:N, :]
    counts_i = counts.astype(jnp.int32)
    pc = (((counts_i + BT - 1) // BT) * BT).astype(jnp.float32)
    er = jax.lax.broadcasted_iota(jnp.int32, (E, E), 0)
    ec = jax.lax.broadcasted_iota(jnp.int32, (E, E), 1)
    mlt = (er < ec).astype(jnp.float32)
    po = jax.lax.dot_general(
        pc, mlt, (((1,), (0,)), ((), ())),
        preferred_element_type=jnp.float32)

    rank0 = jnp.sum(oh0 * cum, axis=1, keepdims=True)
    rank1 = jnp.sum(oh1 * cum, axis=1, keepdims=True)
    off0 = jnp.sum(oh0 * po, axis=1, keepdims=True)
    off1 = jnp.sum(oh1 * po, axis=1, keepdims=True)
    pos0_ref[...] = (off0 + rank0).astype(jnp.int32)
    pos1_ref[...] = (off1 + rank1).astype(jnp.int32)
    w0_ref[...] = w0
    w1_ref[...] = w1

    end = po + pc
    gio = jax.lax.broadcasted_iota(jnp.int32, (32, 1), 0)
    owned = (gio.astype(jnp.float32) * BT >= end)
    be = jnp.sum(owned.astype(jnp.int32), axis=1, keepdims=True)
    be_ref[...] = jnp.minimum(be, E - 1)


def _moe_body(be_ref, xb_ref, p0r_ref, p1r_ref,
              wg_ref, wu_ref, wd_ref, ys_ref):
    g = pl.program_id(0)
    base = g * BT

    pio_c = jax.lax.broadcasted_iota(jnp.int32, (BT, 1), 0) + base
    a0 = (p0r_ref[...] == pio_c)
    a1 = (p1r_ref[...] == pio_c)
    gath = a0.astype(jnp.bfloat16) + a1.astype(jnp.bfloat16)
    rows = jax.lax.dot_general(
        gath, xb_ref[...], (((1,), (0,)), ((), ())),
        preferred_element_type=jnp.float32).astype(jnp.bfloat16)

    y = jnp.zeros((BT, D), dtype=jnp.float32)
    for f in range(2):
        wg_h = wg_ref[0][:, f * FH:(f + 1) * FH].astype(jnp.bfloat16)
        wu_h = wu_ref[0][:, f * FH:(f + 1) * FH].astype(jnp.bfloat16)
        wd_h = wd_ref[0][f * FH:(f + 1) * FH, :].astype(jnp.bfloat16)
        gate = jax.lax.dot_general(
            rows, wg_h, (((1,), (0,)), ((), ())),
            preferred_element_type=jnp.float32)
        up = jax.lax.dot_general(
            rows, wu_h, (((1,), (0,)), ((), ())),
            preferred_element_type=jnp.float32)
        h = (gate * (1.0 / (1.0 + jnp.exp(-gate))) * up).astype(jnp.bfloat16)
        y = y + jax.lax.dot_general(
            h, wd_h, (((1,), (0,)), ((), ())),
            preferred_element_type=jnp.float32)

    ys_ref[...] = y.astype(jnp.bfloat16)


def _comb_body(ys_ref, p0c_ref, p1c_ref, w0c_ref, w1c_ref, out_ref):
    pio_r = jax.lax.broadcasted_iota(jnp.int32, (1, P), 1)
    a0 = (p0c_ref[...] == pio_r)
    a1 = (p1c_ref[...] == pio_r)
    w = (jnp.where(a0, w0c_ref[...], 0.0)
         + jnp.where(a1, w1c_ref[...], 0.0)).astype(jnp.bfloat16)
    out_ref[...] = jax.lax.dot_general(
        w, ys_ref[...], (((1,), (0,)), ((), ())),
        preferred_element_type=jnp.float32)


def kernel(x, router_w, router_b, w_gate, w_up, w_down):
    pos0, pos1, w0, w1, be, x_bf = pl.pallas_call(
        _meta_body,
        out_shape=[
            jax.ShapeDtypeStruct((N, 1), jnp.int32),
            jax.ShapeDtypeStruct((N, 1), jnp.int32),
            jax.ShapeDtypeStruct((N, 1), jnp.float32),
            jax.ShapeDtypeStruct((N, 1), jnp.float32),
            jax.ShapeDtypeStruct((32, 1), jnp.int32),
            jax.ShapeDtypeStruct((N, D), jnp.bfloat16),
        ],
        compiler_params=pltpu.CompilerParams(
            vmem_limit_bytes=128 * 1024 * 1024),
    )(x, router_w, router_b.reshape(1, E))

    be_flat = be.reshape(-1)[:G]
    pos0_r = pos0.reshape(1, N)
    pos1_r = pos1.reshape(1, N)

    moe_spec = pltpu.PrefetchScalarGridSpec(
        num_scalar_prefetch=1,
        grid=(G,),
        in_specs=[
            pl.BlockSpec((N, D), lambda g, be: (0, 0)),
            pl.BlockSpec((1, N), lambda g, be: (0, 0)),
            pl.BlockSpec((1, N), lambda g, be: (0, 0)),
            pl.BlockSpec((1, D, F), lambda g, be: (be[g], 0, 0)),
            pl.BlockSpec((1, D, F), lambda g, be: (be[g], 0, 0)),
            pl.BlockSpec((1, F, D), lambda g, be: (be[g], 0, 0)),
        ],
        out_specs=pl.BlockSpec((BT, D), lambda g, be: (g, 0)),
    )
    y_sorted = pl.pallas_call(
        _moe_body,
        grid_spec=moe_spec,
        out_shape=jax.ShapeDtypeStruct((P, D), jnp.bfloat16),
        compiler_params=pltpu.CompilerParams(
            dimension_semantics=("arbitrary",),
            allow_input_fusion=[True] * 8,
            vmem_limit_bytes=128 * 1024 * 1024),
    )(be_flat, x_bf, pos0_r, pos1_r, w_gate, w_up, w_down)

    out = pl.pallas_call(
        _comb_body,
        grid=(N // BT,),
        in_specs=[
            pl.BlockSpec((P, D), lambda t: (0, 0)),
            pl.BlockSpec((BT, 1), lambda t: (t, 0)),
            pl.BlockSpec((BT, 1), lambda t: (t, 0)),
            pl.BlockSpec((BT, 1), lambda t: (t, 0)),
            pl.BlockSpec((BT, 1), lambda t: (t, 0)),
        ],
        out_specs=pl.BlockSpec((BT, D), lambda t: (t, 0)),
        out_shape=jax.ShapeDtypeStruct((N, D), jnp.float32),
        compiler_params=pltpu.CompilerParams(
            dimension_semantics=("arbitrary",),
            vmem_limit_bytes=128 * 1024 * 1024),
    )(y_sorted, pos0, pos1, w0, w1)
    return out
